# baseline (speedup 1.0000x reference)
import jax
import jax.numpy as jnp
from jax.experimental import pallas as pl
from jax.experimental.pallas import tpu as pltpu

B, S, H, E = 4, 4096, 2048, 8
N = B * S
CH = 2048
NCH = N // CH
EP = 128


def _router_body(x_hbm, wp_ref, logits_ref, idx_ref, w_ref,
                 buf0, buf1, sem0, sem1):
    bufs = (buf0, buf1)
    sems = (sem0, sem1)

    def copy_in(c):
        return pltpu.make_async_copy(
            x_hbm.at[pl.ds(c * CH, CH), :], bufs[c % 2], sems[c % 2])

    copy_in(0).start()
    for c in range(NCH):
        if c + 1 < NCH:
            copy_in(c + 1).start()
        copy_in(c).wait()
        x = bufs[c % 2][...].astype(jnp.bfloat16)
        wp = wp_ref[...].astype(jnp.bfloat16)
        lT = jax.lax.dot_general(wp, x, (((1,), (1,)), ((), ())),
                                 preferred_element_type=jnp.float32)
        logits = lT[:E, :].T
        logits_ref[pl.ds(c * CH, CH), :] = logits
        mx = jnp.max(logits, axis=1, keepdims=True)
        iota = jax.lax.broadcasted_iota(jnp.int32, logits.shape, 1)
        idx = jnp.min(jnp.where(logits == mx, iota, E), axis=1, keepdims=True)
        idx_ref[pl.ds(c * CH, CH), :] = idx
        w_ref[pl.ds(c * CH, CH), :] = jnp.ones_like(mx)


@jax.jit
def kernel(hidden_states, W):
    x = hidden_states.reshape(N, H)
    wp = jnp.zeros((EP, H), jnp.float32).at[:E, :].set(W)
    logits, idx, weights = pl.pallas_call(
        _router_body,
        in_specs=[
            pl.BlockSpec(memory_space=pl.ANY),
            pl.BlockSpec(memory_space=pltpu.MemorySpace.VMEM),
        ],
        out_specs=[
            pl.BlockSpec(memory_space=pltpu.MemorySpace.VMEM),
            pl.BlockSpec(memory_space=pltpu.MemorySpace.VMEM),
            pl.BlockSpec(memory_space=pltpu.MemorySpace.VMEM),
        ],
        out_shape=[
            jax.ShapeDtypeStruct((N, E), jnp.float32),
            jax.ShapeDtypeStruct((N, 1), jnp.int32),
            jax.ShapeDtypeStruct((N, 1), jnp.float32),
        ],
        scratch_shapes=[
            pltpu.MemorySpace.VMEM((CH, H), jnp.float32),
            pltpu.MemorySpace.VMEM((CH, H), jnp.float32),
            pltpu.SemaphoreType.DMA,
            pltpu.SemaphoreType.DMA,
        ],
    )(x, wp)
    return (
        logits.reshape(B, S, E),
        idx.reshape(B, S),
        weights.reshape(B, S),
    )
